# baseline (speedup 1.0000x reference)
import functools

import jax
import jax.numpy as jnp
from jax import lax
from jax.experimental import pallas as pl
from jax.experimental.pallas import tpu as pltpu
from jax.experimental.pallas import tpu_sc as plsc

N = 10000
E = 320000
D = 128
HID = 128
B = 1024

NC, NS, L = 2, 16, 16
NW = NC * NS
BLK = 128
NBLK = 80
EPW = NBLK * BLK
E_PAD = EPW * NW
NPAD = 10240
RPT = NPAD // NS
BATW = B // NW

def _wid():
    return lax.axis_index("s") * NC + lax.axis_index("c")


def _sc_mesh():
    return plsc.VectorSubcoreMesh(core_axis_name="c", subcore_axis_name="s",
                                  num_cores=NC, num_subcores=NS)


@functools.cache
def _make_deg_kernel():
    return pl.kernel(
        _deg_body,
        out_type=jax.ShapeDtypeStruct((NC, NPAD), jnp.float32),
        mesh=_sc_mesh(),
        scratch_types=[
            pltpu.VMEM((BLK,), jnp.int32),
            pltpu.VMEM((BLK,), jnp.float32),
            pltpu.VMEM((RPT,), jnp.float32),
            pltpu.VMEM_SHARED((NPAD,), jnp.float32),
        ],
    )


def _deg_body(dst3_hbm, degp_hbm, didx_v, ones_v, zbuf_v, acc_sh):
    cid = lax.axis_index("c")
    sid = lax.axis_index("s")
    wid = _wid()

    one16 = jnp.full((L,), 1.0, jnp.float32)
    zero16 = jnp.zeros((L,), jnp.float32)

    def fill_ones(i, _):
        ones_v[pl.ds(i * L, L)] = one16
        return 0

    lax.fori_loop(0, BLK // L, fill_ones, 0)

    def fill_zero(i, _):
        zbuf_v[pl.ds(i * L, L)] = zero16
        return 0

    lax.fori_loop(0, RPT // L, fill_zero, 0)
    pltpu.sync_copy(zbuf_v, acc_sh.at[pl.ds(sid * RPT, RPT)])
    plsc.subcore_barrier()

    def body(j, _):
        pltpu.sync_copy(dst3_hbm.at[wid, j], didx_v)
        pltpu.sync_copy(ones_v, acc_sh.at[didx_v], add=True)
        return 0

    lax.fori_loop(0, NBLK, body, 0)
    plsc.subcore_barrier()
    pltpu.sync_copy(acc_sh.at[pl.ds(sid * RPT, RPT)],
                    degp_hbm.at[cid, pl.ds(sid * RPT, RPT)])


@functools.cache
def _make_msgpass_kernel():
    return pl.kernel(
        _msgpass_body,
        out_type=jax.ShapeDtypeStruct((NC, NPAD, D), jnp.float32),
        mesh=_sc_mesh(),
        scratch_types=[
            pltpu.VMEM((BLK,), jnp.int32),
            pltpu.VMEM((BLK,), jnp.int32),
            pltpu.VMEM((BLK,), jnp.int32),
            pltpu.VMEM((BLK,), jnp.int32),
            pltpu.VMEM((BLK, D), jnp.float32),
            pltpu.VMEM((BLK, D), jnp.float32),
            pltpu.SemaphoreType.DMA,
            pltpu.SemaphoreType.DMA,
            pltpu.VMEM_SHARED((NPAD, D), jnp.float32),
        ],
    )


def _msgpass_body(y_hbm, src3_hbm, dst3_hbm, accp_hbm,
                  sidx0_v, didx0_v, sidx1_v, didx1_v,
                  buf0_v, buf1_v, sem0, sem1, acc_sh):
    cid = lax.axis_index("c")
    sid = lax.axis_index("s")
    wid = _wid()

    zero16 = jnp.zeros((L,), jnp.float32)

    def fill_zero(i, _):
        for k in range(D // L):
            buf0_v[i, pl.ds(k * L, L)] = zero16
        return 0

    lax.fori_loop(0, BLK, fill_zero, 0)
    for k in range(RPT // BLK):
        pltpu.sync_copy(buf0_v, acc_sh.at[pl.ds(sid * RPT + k * BLK, BLK)])
    plsc.subcore_barrier()

    def body(j, _):
        pltpu.sync_copy(src3_hbm.at[wid, j], sidx0_v)
        pltpu.sync_copy(dst3_hbm.at[wid, j], didx0_v)
        pltpu.sync_copy(y_hbm.at[sidx0_v], buf0_v)
        pltpu.sync_copy(buf0_v, acc_sh.at[didx0_v], add=True)
        return 0

    lax.fori_loop(0, NBLK, body, 0)
    plsc.subcore_barrier()
    for k in range(RPT // BLK):
        r0 = sid * RPT + k * BLK
        pltpu.sync_copy(acc_sh.at[pl.ds(r0, BLK)],
                        accp_hbm.at[cid, pl.ds(r0, BLK)])


@functools.cache
def _make_pairgather_kernel():
    return pl.kernel(
        _pairgather_body,
        out_type=jax.ShapeDtypeStruct((B, D), jnp.float32),
        mesh=_sc_mesh(),
        scratch_types=[
            pltpu.VMEM((BATW,), jnp.int32),
            pltpu.VMEM((BATW,), jnp.int32),
            pltpu.VMEM((BATW, D), jnp.float32),
            pltpu.VMEM((BATW, D), jnp.float32),
        ],
    )


def _pairgather_body(zw_hbm, zp_hbm, tgt_hbm, srcb_hbm, prod_hbm,
                     tidx_v, sidx_v, bufw_v, bufp_v):
    wid = _wid()
    base = wid * BATW
    pltpu.sync_copy(tgt_hbm.at[pl.ds(base, BATW)], tidx_v)
    pltpu.sync_copy(srcb_hbm.at[pl.ds(base, BATW)], sidx_v)
    pltpu.sync_copy(zw_hbm.at[tidx_v], bufw_v)
    pltpu.sync_copy(zp_hbm.at[sidx_v], bufp_v)

    def mul_row(i, _):
        for k in range(D // L):
            s = pl.ds(k * L, L)
            bufw_v[i, s] = bufw_v[i, s] * bufp_v[i, s]
        return 0

    lax.fori_loop(0, BATW, mul_row, 0)
    pltpu.sync_copy(bufw_v, prod_hbm.at[pl.ds(base, BATW)])


_RB = 1000
_GRID = N // _RB


def _tc_pre_body(degp_ref, x_ref, w1_ref, dinv_ref, y1_ref):
    dp = degp_ref[0] + degp_ref[1] + 1.0
    dinv = lax.rsqrt(jnp.maximum(dp, 1e-12))
    dinv_ref[...] = dinv
    y = jnp.dot(x_ref[...], w1_ref[...], preferred_element_type=jnp.float32)
    y1_ref[...] = y * dinv


def _tc_layer_body(a_ref, y_ref, dinv_ref, b_ref, wn_ref, h_ref, yn_ref):
    dinv = dinv_ref[...]
    h = jnp.maximum(dinv * (a_ref[0] + a_ref[1] + y_ref[...]) + b_ref[...], 0.0)
    h_ref[...] = h
    yn = jnp.dot(h, wn_ref[...], preferred_element_type=jnp.float32)
    yn_ref[...] = yn * dinv


def _tc_post_body(a_ref, y_ref, dinv_ref, b3_ref, h1_ref, h2_ref,
                  ww_ref, bw_ref, wp_ref, bp_ref, zw_ref, zp_ref):
    dinv = dinv_ref[...]
    h3 = jnp.maximum(dinv * (a_ref[0] + a_ref[1] + y_ref[...]) + b3_ref[...], 0.0)
    h1 = h1_ref[...]
    h2 = h2_ref[...]
    f32 = jnp.float32
    zw = (jnp.dot(h1, ww_ref[0:HID], preferred_element_type=f32)
          + jnp.dot(h2, ww_ref[HID:2 * HID], preferred_element_type=f32)
          + jnp.dot(h3, ww_ref[2 * HID:], preferred_element_type=f32)
          + bw_ref[...])
    zp = (jnp.dot(h1, wp_ref[0:HID], preferred_element_type=f32)
          + jnp.dot(h2, wp_ref[HID:2 * HID], preferred_element_type=f32)
          + jnp.dot(h3, wp_ref[2 * HID:], preferred_element_type=f32)
          + bp_ref[...])
    zw_ref[...] = zw
    zp_ref[...] = zp


def _tc_rowsum_body(prod_ref, out_ref):
    out_ref[...] = jnp.sum(prod_ref[...], axis=1, keepdims=True)


def _rb_spec(nd=2):
    if nd == 2:
        return pl.BlockSpec((_RB, D), lambda i: (i, 0))
    return pl.BlockSpec((NC, _RB, D), lambda i: (0, i, 0))


_dinv_spec = pl.BlockSpec((_RB, 1), lambda i: (i, 0))
_bias_spec = pl.BlockSpec((1, HID), lambda i: (0, 0))


def kernel(x, edge_index, src_idx, tgt_idx, W1, b1, W2, b2, W3, b3, Ww, bw, Wp, bp):
    src = edge_index[0]
    dst = edge_index[1]
    pad = E_PAD - E
    src_p = jnp.concatenate([src, jnp.zeros((pad,), jnp.int32)])
    dst_p = jnp.concatenate([dst, jnp.full((pad,), N, jnp.int32)])
    src3 = src_p.reshape(NW, NBLK, BLK)
    dst3 = dst_p.reshape(NW, NBLK, BLK)

    degp = _make_deg_kernel()(dst3)
    degp3 = degp[:, :N].reshape(NC, N, 1)

    dinv, y1 = pl.pallas_call(
        _tc_pre_body,
        grid=(_GRID,),
        in_specs=[pl.BlockSpec((NC, _RB, 1), lambda i: (0, i, 0)),
                  _rb_spec(), pl.BlockSpec((D, HID), lambda i: (0, 0))],
        out_specs=[_dinv_spec, _rb_spec()],
        out_shape=[jax.ShapeDtypeStruct((N, 1), jnp.float32),
                   jax.ShapeDtypeStruct((N, HID), jnp.float32)],
    )(degp3, x, W1)

    hs = []
    y = y1
    for (bcur, Wn) in ((b1, W2), (b2, W3)):
        accp = _make_msgpass_kernel()(y, src3, dst3)
        a = accp[:, :N, :]
        h, y = pl.pallas_call(
            _tc_layer_body,
            grid=(_GRID,),
            in_specs=[_rb_spec(3), _rb_spec(), _dinv_spec, _bias_spec,
                      pl.BlockSpec((HID, HID), lambda i: (0, 0))],
            out_specs=[_rb_spec(), _rb_spec()],
            out_shape=[jax.ShapeDtypeStruct((N, HID), jnp.float32),
                       jax.ShapeDtypeStruct((N, HID), jnp.float32)],
        )(a, y, dinv, bcur.reshape(1, HID), Wn)
        hs.append(h)

    accp = _make_msgpass_kernel()(y, src3, dst3)
    a = accp[:, :N, :]
    zw, zp = pl.pallas_call(
        _tc_post_body,
        grid=(_GRID,),
        in_specs=[_rb_spec(3), _rb_spec(), _dinv_spec, _bias_spec,
                  _rb_spec(), _rb_spec(),
                  pl.BlockSpec((3 * HID, HID), lambda i: (0, 0)), _bias_spec,
                  pl.BlockSpec((3 * HID, HID), lambda i: (0, 0)), _bias_spec],
        out_specs=[_rb_spec(), _rb_spec()],
        out_shape=[jax.ShapeDtypeStruct((N, HID), jnp.float32),
                   jax.ShapeDtypeStruct((N, HID), jnp.float32)],
    )(a, y, dinv, b3.reshape(1, HID), hs[0], hs[1],
      Ww, bw.reshape(1, HID), Wp, bp.reshape(1, HID))

    prod = _make_pairgather_kernel()(zw, zp, tgt_idx, src_idx)
    out = pl.pallas_call(
        _tc_rowsum_body,
        grid=(1,),
        in_specs=[pl.BlockSpec((B, D), lambda i: (0, 0))],
        out_specs=pl.BlockSpec((B, 1), lambda i: (0, 0)),
        out_shape=jax.ShapeDtypeStruct((B, 1), jnp.float32),
    )(prod)
    return out.reshape(B)


# speedup vs baseline: 1.0003x; 1.0003x over previous
import functools

import jax
import jax.numpy as jnp
from jax import lax
from jax.experimental import pallas as pl
from jax.experimental.pallas import tpu as pltpu
from jax.experimental.pallas import tpu_sc as plsc

N = 10000
E = 320000
D = 128
HID = 128
B = 1024

NC, NS, L = 2, 16, 16
NW = NC * NS
BLK = 128
NBLK = 80
EPW = NBLK * BLK
E_PAD = EPW * NW
NPAD = 10240
RPT = NPAD // NS
BATW = B // NW

def _wid():
    return lax.axis_index("s") * NC + lax.axis_index("c")


def _sc_mesh():
    return plsc.VectorSubcoreMesh(core_axis_name="c", subcore_axis_name="s",
                                  num_cores=NC, num_subcores=NS)


@functools.cache
def _make_deg_kernel():
    return pl.kernel(
        _deg_body,
        out_type=jax.ShapeDtypeStruct((NC, NPAD), jnp.float32),
        mesh=_sc_mesh(),
        scratch_types=[
            pltpu.VMEM((BLK,), jnp.int32),
            pltpu.VMEM((BLK,), jnp.float32),
            pltpu.VMEM((RPT,), jnp.float32),
            pltpu.VMEM_SHARED((NPAD,), jnp.float32),
        ],
    )


def _deg_body(dst_hbm, degp_hbm, didx_v, ones_v, zbuf_v, acc_sh):
    cid = lax.axis_index("c")
    sid = lax.axis_index("s")
    wid = _wid()

    one16 = jnp.full((L,), 1.0, jnp.float32)
    zero16 = jnp.zeros((L,), jnp.float32)

    def fill_ones(i, _):
        ones_v[pl.ds(i * L, L)] = one16
        return 0

    lax.fori_loop(0, BLK // L, fill_ones, 0)

    def fill_zero(i, _):
        zbuf_v[pl.ds(i * L, L)] = zero16
        return 0

    lax.fori_loop(0, RPT // L, fill_zero, 0)
    pltpu.sync_copy(zbuf_v, acc_sh.at[pl.ds(sid * RPT, RPT)])
    plsc.subcore_barrier()

    base = wid * EPW

    def body(j, _):
        pltpu.sync_copy(dst_hbm.at[pl.ds(base + j * BLK, BLK)], didx_v)
        pltpu.sync_copy(ones_v, acc_sh.at[didx_v], add=True)
        return 0

    lax.fori_loop(0, NBLK, body, 0)
    plsc.subcore_barrier()
    pltpu.sync_copy(acc_sh.at[pl.ds(sid * RPT, RPT)],
                    degp_hbm.at[cid, pl.ds(sid * RPT, RPT)])


@functools.cache
def _make_msgpass_kernel():
    return pl.kernel(
        _msgpass_body,
        out_type=jax.ShapeDtypeStruct((NC, NPAD, D), jnp.float32),
        mesh=_sc_mesh(),
        scratch_types=[
            pltpu.VMEM((BLK,), jnp.int32),
            pltpu.VMEM((BLK,), jnp.int32),
            pltpu.VMEM((BLK,), jnp.int32),
            pltpu.VMEM((BLK,), jnp.int32),
            pltpu.VMEM((BLK, D), jnp.float32),
            pltpu.VMEM((BLK, D), jnp.float32),
            pltpu.SemaphoreType.DMA,
            pltpu.SemaphoreType.DMA,
            pltpu.VMEM_SHARED((NPAD, D), jnp.float32),
        ],
    )


def _msgpass_body(y_hbm, src_hbm, dst_hbm, accp_hbm,
                  sidx0_v, didx0_v, sidx1_v, didx1_v,
                  buf0_v, buf1_v, sem0, sem1, acc_sh):
    cid = lax.axis_index("c")
    sid = lax.axis_index("s")
    wid = _wid()

    zero16 = jnp.zeros((L,), jnp.float32)

    def fill_zero(i, _):
        for k in range(D // L):
            buf0_v[i, pl.ds(k * L, L)] = zero16
        return 0

    lax.fori_loop(0, BLK, fill_zero, 0)
    for k in range(RPT // BLK):
        pltpu.sync_copy(buf0_v, acc_sh.at[pl.ds(sid * RPT + k * BLK, BLK)])
    plsc.subcore_barrier()

    base = wid * EPW

    def body(j, _):
        pltpu.sync_copy(src_hbm.at[pl.ds(base + j * BLK, BLK)], sidx0_v)
        pltpu.sync_copy(dst_hbm.at[pl.ds(base + j * BLK, BLK)], didx0_v)
        pltpu.sync_copy(y_hbm.at[sidx0_v], buf0_v)
        pltpu.sync_copy(buf0_v, acc_sh.at[didx0_v], add=True)
        return 0

    lax.fori_loop(0, NBLK, body, 0)
    plsc.subcore_barrier()
    for k in range(RPT // BLK):
        r0 = sid * RPT + k * BLK
        pltpu.sync_copy(acc_sh.at[pl.ds(r0, BLK)],
                        accp_hbm.at[cid, pl.ds(r0, BLK)])


@functools.cache
def _make_pairgather_kernel():
    return pl.kernel(
        _pairgather_body,
        out_type=jax.ShapeDtypeStruct((B, D), jnp.float32),
        mesh=_sc_mesh(),
        scratch_types=[
            pltpu.VMEM((BATW,), jnp.int32),
            pltpu.VMEM((BATW,), jnp.int32),
            pltpu.VMEM((BATW, D), jnp.float32),
            pltpu.VMEM((BATW, D), jnp.float32),
        ],
    )


def _pairgather_body(zw_hbm, zp_hbm, tgt_hbm, srcb_hbm, prod_hbm,
                     tidx_v, sidx_v, bufw_v, bufp_v):
    wid = _wid()
    base = wid * BATW
    pltpu.sync_copy(tgt_hbm.at[pl.ds(base, BATW)], tidx_v)
    pltpu.sync_copy(srcb_hbm.at[pl.ds(base, BATW)], sidx_v)
    pltpu.sync_copy(zw_hbm.at[tidx_v], bufw_v)
    pltpu.sync_copy(zp_hbm.at[sidx_v], bufp_v)

    def mul_row(i, _):
        for k in range(D // L):
            s = pl.ds(k * L, L)
            bufw_v[i, s] = bufw_v[i, s] * bufp_v[i, s]
        return 0

    lax.fori_loop(0, BATW, mul_row, 0)
    pltpu.sync_copy(bufw_v, prod_hbm.at[pl.ds(base, BATW)])


_RB = 1000
_GRID = N // _RB


def _tc_pre_body(degp_ref, x_ref, w1_ref, dinv_ref, y1_ref):
    dp = degp_ref[0] + degp_ref[1] + 1.0
    dinv = lax.rsqrt(jnp.maximum(dp, 1e-12))
    dinv_ref[...] = dinv
    y = jnp.dot(x_ref[...], w1_ref[...], preferred_element_type=jnp.float32)
    y1_ref[...] = y * dinv


def _tc_layer_body(a_ref, y_ref, dinv_ref, b_ref, wn_ref, h_ref, yn_ref):
    dinv = dinv_ref[...]
    h = jnp.maximum(dinv * (a_ref[0] + a_ref[1] + y_ref[...]) + b_ref[...], 0.0)
    h_ref[...] = h
    yn = jnp.dot(h, wn_ref[...], preferred_element_type=jnp.float32)
    yn_ref[...] = yn * dinv


def _tc_post_body(a_ref, y_ref, dinv_ref, b3_ref, h1_ref, h2_ref,
                  ww_ref, bw_ref, wp_ref, bp_ref, zw_ref, zp_ref):
    dinv = dinv_ref[...]
    h3 = jnp.maximum(dinv * (a_ref[0] + a_ref[1] + y_ref[...]) + b3_ref[...], 0.0)
    h1 = h1_ref[...]
    h2 = h2_ref[...]
    f32 = jnp.float32
    zw = (jnp.dot(h1, ww_ref[0:HID], preferred_element_type=f32)
          + jnp.dot(h2, ww_ref[HID:2 * HID], preferred_element_type=f32)
          + jnp.dot(h3, ww_ref[2 * HID:], preferred_element_type=f32)
          + bw_ref[...])
    zp = (jnp.dot(h1, wp_ref[0:HID], preferred_element_type=f32)
          + jnp.dot(h2, wp_ref[HID:2 * HID], preferred_element_type=f32)
          + jnp.dot(h3, wp_ref[2 * HID:], preferred_element_type=f32)
          + bp_ref[...])
    zw_ref[...] = zw
    zp_ref[...] = zp


def _tc_rowsum_body(prod_ref, out_ref):
    out_ref[...] = jnp.sum(prod_ref[...], axis=1, keepdims=True)


def _rb_spec(nd=2):
    if nd == 2:
        return pl.BlockSpec((_RB, D), lambda i: (i, 0))
    return pl.BlockSpec((NC, _RB, D), lambda i: (0, i, 0))


_dinv_spec = pl.BlockSpec((_RB, 1), lambda i: (i, 0))
_bias_spec = pl.BlockSpec((1, HID), lambda i: (0, 0))


def kernel(x, edge_index, src_idx, tgt_idx, W1, b1, W2, b2, W3, b3, Ww, bw, Wp, bp):
    src = edge_index[0]
    dst = edge_index[1]
    pad = E_PAD - E
    src_p = jnp.concatenate([src, jnp.zeros((pad,), jnp.int32)])
    dst_p = jnp.concatenate([dst, jnp.full((pad,), N, jnp.int32)])

    degp = _make_deg_kernel()(dst_p)
    degp3 = degp[:, :N].reshape(NC, N, 1)

    dinv, y1 = pl.pallas_call(
        _tc_pre_body,
        grid=(_GRID,),
        in_specs=[pl.BlockSpec((NC, _RB, 1), lambda i: (0, i, 0)),
                  _rb_spec(), pl.BlockSpec((D, HID), lambda i: (0, 0))],
        out_specs=[_dinv_spec, _rb_spec()],
        out_shape=[jax.ShapeDtypeStruct((N, 1), jnp.float32),
                   jax.ShapeDtypeStruct((N, HID), jnp.float32)],
    )(degp3, x, W1)

    hs = []
    y = y1
    for (bcur, Wn) in ((b1, W2), (b2, W3)):
        accp = _make_msgpass_kernel()(y, src_p, dst_p)
        a = accp[:, :N, :]
        h, y = pl.pallas_call(
            _tc_layer_body,
            grid=(_GRID,),
            in_specs=[_rb_spec(3), _rb_spec(), _dinv_spec, _bias_spec,
                      pl.BlockSpec((HID, HID), lambda i: (0, 0))],
            out_specs=[_rb_spec(), _rb_spec()],
            out_shape=[jax.ShapeDtypeStruct((N, HID), jnp.float32),
                       jax.ShapeDtypeStruct((N, HID), jnp.float32)],
        )(a, y, dinv, bcur.reshape(1, HID), Wn)
        hs.append(h)

    accp = _make_msgpass_kernel()(y, src_p, dst_p)
    a = accp[:, :N, :]
    zw, zp = pl.pallas_call(
        _tc_post_body,
        grid=(_GRID,),
        in_specs=[_rb_spec(3), _rb_spec(), _dinv_spec, _bias_spec,
                  _rb_spec(), _rb_spec(),
                  pl.BlockSpec((3 * HID, HID), lambda i: (0, 0)), _bias_spec,
                  pl.BlockSpec((3 * HID, HID), lambda i: (0, 0)), _bias_spec],
        out_specs=[_rb_spec(), _rb_spec()],
        out_shape=[jax.ShapeDtypeStruct((N, HID), jnp.float32),
                   jax.ShapeDtypeStruct((N, HID), jnp.float32)],
    )(a, y, dinv, b3.reshape(1, HID), hs[0], hs[1],
      Ww, bw.reshape(1, HID), Wp, bp.reshape(1, HID))

    prod = _make_pairgather_kernel()(zw, zp, tgt_idx, src_idx)
    out = pl.pallas_call(
        _tc_rowsum_body,
        grid=(1,),
        in_specs=[pl.BlockSpec((B, D), lambda i: (0, 0))],
        out_specs=pl.BlockSpec((B, 1), lambda i: (0, 0)),
        out_shape=jax.ShapeDtypeStruct((B, 1), jnp.float32),
    )(prod)
    return out.reshape(B)


# speedup vs baseline: 1.0004x; 1.0001x over previous
import functools

import jax
import jax.numpy as jnp
from jax import lax
from jax.experimental import pallas as pl
from jax.experimental.pallas import tpu as pltpu
from jax.experimental.pallas import tpu_sc as plsc

N = 10000
E = 320000
D = 128
HID = 128
B = 1024

NC, NS, L = 2, 16, 16
NW = NC * NS
BLK = 128
NBLK = 80
EPW = NBLK * BLK
E_PAD = EPW * NW
NPAD = 10240
RPT = NPAD // NS
BATW = B // NW

def _wid():
    return lax.axis_index("s") * NC + lax.axis_index("c")


def _sc_mesh():
    return plsc.VectorSubcoreMesh(core_axis_name="c", subcore_axis_name="s",
                                  num_cores=NC, num_subcores=NS)


@functools.cache
def _make_deg_kernel():
    return pl.kernel(
        _deg_body,
        out_type=jax.ShapeDtypeStruct((NC, NPAD), jnp.float32),
        mesh=_sc_mesh(),
        scratch_types=[
            pltpu.VMEM((BLK,), jnp.int32),
            pltpu.VMEM((BLK,), jnp.float32),
            pltpu.VMEM((RPT,), jnp.float32),
            pltpu.VMEM_SHARED((NPAD,), jnp.float32),
        ],
    )


def _deg_body(dst_hbm, degp_hbm, didx_v, ones_v, zbuf_v, acc_sh):
    cid = lax.axis_index("c")
    sid = lax.axis_index("s")
    wid = _wid()

    one16 = jnp.full((L,), 1.0, jnp.float32)
    zero16 = jnp.zeros((L,), jnp.float32)

    def fill_ones(i, _):
        ones_v[pl.ds(i * L, L)] = one16
        return 0

    lax.fori_loop(0, BLK // L, fill_ones, 0)

    def fill_zero(i, _):
        zbuf_v[pl.ds(i * L, L)] = zero16
        return 0

    lax.fori_loop(0, RPT // L, fill_zero, 0)
    pltpu.sync_copy(zbuf_v, acc_sh.at[pl.ds(sid * RPT, RPT)])
    plsc.subcore_barrier()

    base = wid * EPW

    def body(j, _):
        pltpu.sync_copy(dst_hbm.at[pl.ds(base + j * BLK, BLK)], didx_v)
        pltpu.sync_copy(ones_v, acc_sh.at[didx_v], add=True)
        return 0

    lax.fori_loop(0, NBLK, body, 0)
    plsc.subcore_barrier()
    pltpu.sync_copy(acc_sh.at[pl.ds(sid * RPT, RPT)],
                    degp_hbm.at[cid, pl.ds(sid * RPT, RPT)])


@functools.cache
def _make_msgpass_kernel():
    return pl.kernel(
        _msgpass_body,
        out_type=jax.ShapeDtypeStruct((NC, NPAD, D), jnp.float32),
        mesh=_sc_mesh(),
        scratch_types=[
            pltpu.VMEM((BLK,), jnp.int32),
            pltpu.VMEM((BLK,), jnp.int32),
            pltpu.VMEM((BLK, D), jnp.float32),
            pltpu.VMEM_SHARED((NPAD, D), jnp.float32),
        ],
    )


def _msgpass_body(y_hbm, src_hbm, dst_hbm, accp_hbm,
                  sidx0_v, didx0_v, buf0_v, acc_sh):
    cid = lax.axis_index("c")
    sid = lax.axis_index("s")
    wid = _wid()

    zero16 = jnp.zeros((L,), jnp.float32)

    def fill_zero(i, _):
        for k in range(D // L):
            buf0_v[i, pl.ds(k * L, L)] = zero16
        return 0

    lax.fori_loop(0, BLK, fill_zero, 0)
    for k in range(RPT // BLK):
        pltpu.sync_copy(buf0_v, acc_sh.at[pl.ds(sid * RPT + k * BLK, BLK)])
    plsc.subcore_barrier()

    base = wid * EPW

    def body(j, _):
        pltpu.sync_copy(src_hbm.at[pl.ds(base + j * BLK, BLK)], sidx0_v)
        pltpu.sync_copy(dst_hbm.at[pl.ds(base + j * BLK, BLK)], didx0_v)
        pltpu.sync_copy(y_hbm.at[sidx0_v], buf0_v)
        pltpu.sync_copy(buf0_v, acc_sh.at[didx0_v], add=True)
        return 0

    lax.fori_loop(0, NBLK, body, 0)
    plsc.subcore_barrier()
    for k in range(RPT // BLK):
        r0 = sid * RPT + k * BLK
        pltpu.sync_copy(acc_sh.at[pl.ds(r0, BLK)],
                        accp_hbm.at[cid, pl.ds(r0, BLK)])


@functools.cache
def _make_pairgather_kernel():
    return pl.kernel(
        _pairgather_body,
        out_type=jax.ShapeDtypeStruct((B, D), jnp.float32),
        mesh=_sc_mesh(),
        scratch_types=[
            pltpu.VMEM((BATW,), jnp.int32),
            pltpu.VMEM((BATW,), jnp.int32),
            pltpu.VMEM((BATW, D), jnp.float32),
            pltpu.VMEM((BATW, D), jnp.float32),
        ],
    )


def _pairgather_body(zw_hbm, zp_hbm, tgt_hbm, srcb_hbm, prod_hbm,
                     tidx_v, sidx_v, bufw_v, bufp_v):
    wid = _wid()
    base = wid * BATW
    pltpu.sync_copy(tgt_hbm.at[pl.ds(base, BATW)], tidx_v)
    pltpu.sync_copy(srcb_hbm.at[pl.ds(base, BATW)], sidx_v)
    pltpu.sync_copy(zw_hbm.at[tidx_v], bufw_v)
    pltpu.sync_copy(zp_hbm.at[sidx_v], bufp_v)

    def mul_row(i, _):
        for k in range(D // L):
            s = pl.ds(k * L, L)
            bufw_v[i, s] = bufw_v[i, s] * bufp_v[i, s]
        return 0

    lax.fori_loop(0, BATW, mul_row, 0)
    pltpu.sync_copy(bufw_v, prod_hbm.at[pl.ds(base, BATW)])


_RB = 1000
_GRID = N // _RB


def _tc_pre_body(degp_ref, x_ref, w1_ref, dinv_ref, y1_ref):
    dp = degp_ref[0] + degp_ref[1] + 1.0
    dinv = lax.rsqrt(jnp.maximum(dp, 1e-12))
    dinv_ref[...] = dinv
    y = jnp.dot(x_ref[...], w1_ref[...], preferred_element_type=jnp.float32)
    y1_ref[...] = y * dinv


def _tc_layer_body(a_ref, y_ref, dinv_ref, b_ref, wn_ref, h_ref, yn_ref):
    dinv = dinv_ref[...]
    h = jnp.maximum(dinv * (a_ref[0] + a_ref[1] + y_ref[...]) + b_ref[...], 0.0)
    h_ref[...] = h
    yn = jnp.dot(h, wn_ref[...], preferred_element_type=jnp.float32)
    yn_ref[...] = yn * dinv


def _tc_post_body(a_ref, y_ref, dinv_ref, b3_ref, h1_ref, h2_ref,
                  ww_ref, bw_ref, wp_ref, bp_ref, zw_ref, zp_ref):
    dinv = dinv_ref[...]
    h3 = jnp.maximum(dinv * (a_ref[0] + a_ref[1] + y_ref[...]) + b3_ref[...], 0.0)
    h1 = h1_ref[...]
    h2 = h2_ref[...]
    f32 = jnp.float32
    zw = (jnp.dot(h1, ww_ref[0:HID], preferred_element_type=f32)
          + jnp.dot(h2, ww_ref[HID:2 * HID], preferred_element_type=f32)
          + jnp.dot(h3, ww_ref[2 * HID:], preferred_element_type=f32)
          + bw_ref[...])
    zp = (jnp.dot(h1, wp_ref[0:HID], preferred_element_type=f32)
          + jnp.dot(h2, wp_ref[HID:2 * HID], preferred_element_type=f32)
          + jnp.dot(h3, wp_ref[2 * HID:], preferred_element_type=f32)
          + bp_ref[...])
    zw_ref[...] = zw
    zp_ref[...] = zp


def _tc_rowsum_body(prod_ref, out_ref):
    out_ref[...] = jnp.sum(prod_ref[...], axis=1, keepdims=True)


def _rb_spec(nd=2):
    if nd == 2:
        return pl.BlockSpec((_RB, D), lambda i: (i, 0))
    return pl.BlockSpec((NC, _RB, D), lambda i: (0, i, 0))


_dinv_spec = pl.BlockSpec((_RB, 1), lambda i: (i, 0))
_bias_spec = pl.BlockSpec((1, HID), lambda i: (0, 0))


def kernel(x, edge_index, src_idx, tgt_idx, W1, b1, W2, b2, W3, b3, Ww, bw, Wp, bp):
    src = edge_index[0]
    dst = edge_index[1]
    pad = E_PAD - E
    src_p = jnp.concatenate([src, jnp.zeros((pad,), jnp.int32)])
    dst_p = jnp.concatenate([dst, jnp.full((pad,), N, jnp.int32)])

    degp = _make_deg_kernel()(dst_p)
    degp3 = degp[:, :N].reshape(NC, N, 1)

    dinv, y1 = pl.pallas_call(
        _tc_pre_body,
        grid=(_GRID,),
        in_specs=[pl.BlockSpec((NC, _RB, 1), lambda i: (0, i, 0)),
                  _rb_spec(), pl.BlockSpec((D, HID), lambda i: (0, 0))],
        out_specs=[_dinv_spec, _rb_spec()],
        out_shape=[jax.ShapeDtypeStruct((N, 1), jnp.float32),
                   jax.ShapeDtypeStruct((N, HID), jnp.float32)],
    )(degp3, x, W1)

    hs = []
    y = y1
    for (bcur, Wn) in ((b1, W2), (b2, W3)):
        accp = _make_msgpass_kernel()(y, src_p, dst_p)
        a = accp[:, :N, :]
        h, y = pl.pallas_call(
            _tc_layer_body,
            grid=(_GRID,),
            in_specs=[_rb_spec(3), _rb_spec(), _dinv_spec, _bias_spec,
                      pl.BlockSpec((HID, HID), lambda i: (0, 0))],
            out_specs=[_rb_spec(), _rb_spec()],
            out_shape=[jax.ShapeDtypeStruct((N, HID), jnp.float32),
                       jax.ShapeDtypeStruct((N, HID), jnp.float32)],
        )(a, y, dinv, bcur.reshape(1, HID), Wn)
        hs.append(h)

    accp = _make_msgpass_kernel()(y, src_p, dst_p)
    a = accp[:, :N, :]
    zw, zp = pl.pallas_call(
        _tc_post_body,
        grid=(_GRID,),
        in_specs=[_rb_spec(3), _rb_spec(), _dinv_spec, _bias_spec,
                  _rb_spec(), _rb_spec(),
                  pl.BlockSpec((3 * HID, HID), lambda i: (0, 0)), _bias_spec,
                  pl.BlockSpec((3 * HID, HID), lambda i: (0, 0)), _bias_spec],
        out_specs=[_rb_spec(), _rb_spec()],
        out_shape=[jax.ShapeDtypeStruct((N, HID), jnp.float32),
                   jax.ShapeDtypeStruct((N, HID), jnp.float32)],
    )(a, y, dinv, b3.reshape(1, HID), hs[0], hs[1],
      Ww, bw.reshape(1, HID), Wp, bp.reshape(1, HID))

    prod = _make_pairgather_kernel()(zw, zp, tgt_idx, src_idx)
    out = pl.pallas_call(
        _tc_rowsum_body,
        grid=(1,),
        in_specs=[pl.BlockSpec((B, D), lambda i: (0, 0))],
        out_specs=pl.BlockSpec((B, 1), lambda i: (0, 0)),
        out_shape=jax.ShapeDtypeStruct((B, 1), jnp.float32),
    )(prod)
    return out.reshape(B)


# speedup vs baseline: 2.2401x; 2.2391x over previous
import functools

import jax
import jax.numpy as jnp
from jax import lax
from jax.experimental import pallas as pl
from jax.experimental.pallas import tpu as pltpu
from jax.experimental.pallas import tpu_sc as plsc

N = 10000
E = 320000
D = 128
HID = 128
B = 1024

NC, NS, L = 2, 16, 16
NW = NC * NS
BLK = 128
NBLK = 80
EPW = NBLK * BLK
E_PAD = EPW * NW
NPAD = 10240
RPT = NPAD // NS
BATW = B // NW

def _wid():
    return lax.axis_index("s") * NC + lax.axis_index("c")


def _sc_mesh():
    return plsc.VectorSubcoreMesh(core_axis_name="c", subcore_axis_name="s",
                                  num_cores=NC, num_subcores=NS)


@functools.cache
def _make_deg_kernel():
    return pl.kernel(
        _deg_body,
        out_type=jax.ShapeDtypeStruct((NC, NPAD), jnp.float32),
        mesh=_sc_mesh(),
        scratch_types=[
            pltpu.VMEM((BLK,), jnp.int32),
            pltpu.VMEM((BLK,), jnp.float32),
            pltpu.VMEM((RPT,), jnp.float32),
            pltpu.VMEM_SHARED((NPAD,), jnp.float32),
        ],
    )


def _deg_body(dst_hbm, degp_hbm, didx_v, ones_v, zbuf_v, acc_sh):
    cid = lax.axis_index("c")
    sid = lax.axis_index("s")
    wid = _wid()

    one16 = jnp.full((L,), 1.0, jnp.float32)
    zero16 = jnp.zeros((L,), jnp.float32)

    def fill_ones(i, _):
        ones_v[pl.ds(i * L, L)] = one16
        return 0

    lax.fori_loop(0, BLK // L, fill_ones, 0)

    def fill_zero(i, _):
        zbuf_v[pl.ds(i * L, L)] = zero16
        return 0

    lax.fori_loop(0, RPT // L, fill_zero, 0)
    pltpu.sync_copy(zbuf_v, acc_sh.at[pl.ds(sid * RPT, RPT)])
    plsc.subcore_barrier()

    base = wid * EPW

    def body(j, _):
        pltpu.sync_copy(dst_hbm.at[pl.ds(base + j * BLK, BLK)], didx_v)
        pltpu.sync_copy(ones_v, acc_sh.at[didx_v], add=True)
        return 0

    lax.fori_loop(0, NBLK, body, 0)
    plsc.subcore_barrier()
    pltpu.sync_copy(acc_sh.at[pl.ds(sid * RPT, RPT)],
                    degp_hbm.at[cid, pl.ds(sid * RPT, RPT)])


@functools.cache
def _make_msgpass_kernel():
    return pl.kernel(
        _msgpass_body,
        out_type=jax.ShapeDtypeStruct((NC, NPAD, D), jnp.float32),
        mesh=_sc_mesh(),
        scratch_types=[
            pltpu.VMEM((BLK,), jnp.int32),
            pltpu.VMEM((BLK,), jnp.int32),
            pltpu.VMEM((BLK, D), jnp.float32),
            pltpu.VMEM_SHARED((NPAD, D), jnp.float32),
        ],
    )


def _msgpass_body(y_hbm, src_hbm, dst_hbm, accp_hbm,
                  sidx0_v, didx0_v, buf0_v, acc_sh):
    cid = lax.axis_index("c")
    sid = lax.axis_index("s")
    wid = _wid()

    zero16 = jnp.zeros((L,), jnp.float32)

    def fill_zero(i, _):
        for k in range(D // L):
            buf0_v[i, pl.ds(k * L, L)] = zero16
        return 0

    lax.fori_loop(0, BLK, fill_zero, 0)
    for k in range(RPT // BLK):
        pltpu.sync_copy(buf0_v, acc_sh.at[pl.ds(sid * RPT + k * BLK, BLK)])
    plsc.subcore_barrier()

    base = wid * EPW

    def body(j, _):
        pltpu.sync_copy(src_hbm.at[pl.ds(base + j * BLK, BLK)], sidx0_v)
        pltpu.sync_copy(dst_hbm.at[pl.ds(base + j * BLK, BLK)], didx0_v)
        pltpu.sync_copy(y_hbm.at[sidx0_v], buf0_v)
        pltpu.sync_copy(buf0_v, acc_sh.at[didx0_v], add=True)
        return 0

    lax.fori_loop(0, NBLK, body, 0)
    plsc.subcore_barrier()
    for k in range(RPT // BLK):
        r0 = sid * RPT + k * BLK
        pltpu.sync_copy(acc_sh.at[pl.ds(r0, BLK)],
                        accp_hbm.at[cid, pl.ds(r0, BLK)])


@functools.cache
def _make_pairgather_kernel():
    return pl.kernel(
        _pairgather_body,
        out_type=jax.ShapeDtypeStruct((B, D), jnp.float32),
        mesh=_sc_mesh(),
        scratch_types=[
            pltpu.VMEM((BATW,), jnp.int32),
            pltpu.VMEM((BATW,), jnp.int32),
            pltpu.VMEM((BATW, D), jnp.float32),
            pltpu.VMEM((BATW, D), jnp.float32),
        ],
    )


def _pairgather_body(zw_hbm, zp_hbm, tgt_hbm, srcb_hbm, prod_hbm,
                     tidx_v, sidx_v, bufw_v, bufp_v):
    wid = _wid()
    base = wid * BATW
    pltpu.sync_copy(tgt_hbm.at[pl.ds(base, BATW)], tidx_v)
    pltpu.sync_copy(srcb_hbm.at[pl.ds(base, BATW)], sidx_v)
    pltpu.sync_copy(zw_hbm.at[tidx_v], bufw_v)
    pltpu.sync_copy(zp_hbm.at[sidx_v], bufp_v)

    def mul_row(i, _):
        for k in range(D // L):
            s = pl.ds(k * L, L)
            bufw_v[i, s] = bufw_v[i, s] * bufp_v[i, s]
        return 0

    lax.fori_loop(0, BATW, mul_row, 0)
    pltpu.sync_copy(bufw_v, prod_hbm.at[pl.ds(base, BATW)])


_RB = 1000
_GRID = N // _RB


def _tc_pre_body(degp_ref, x_ref, w1_ref, dinv_ref, y1_ref):
    dp = degp_ref[0] + degp_ref[1] + 1.0
    dinv = lax.rsqrt(jnp.maximum(dp, 1e-12))
    dinv_ref[...] = dinv
    y = jnp.dot(x_ref[...], w1_ref[...], preferred_element_type=jnp.float32)
    y1_ref[...] = y * dinv


def _tc_layer_body(a_ref, y_ref, dinv_ref, b_ref, wn_ref, h_ref, yn_ref):
    dinv = dinv_ref[...]
    h = jnp.maximum(dinv * (a_ref[0] + a_ref[1] + y_ref[...]) + b_ref[...], 0.0)
    h_ref[...] = h
    yn = jnp.dot(h, wn_ref[...], preferred_element_type=jnp.float32)
    yn_ref[...] = yn * dinv


def _tc_post_body(a_ref, y_ref, dinv_ref, b3_ref, h1_ref, h2_ref,
                  ww_ref, bw_ref, wp_ref, bp_ref, zw_ref, zp_ref):
    dinv = dinv_ref[...]
    h3 = jnp.maximum(dinv * (a_ref[0] + a_ref[1] + y_ref[...]) + b3_ref[...], 0.0)
    h1 = h1_ref[...]
    h2 = h2_ref[...]
    f32 = jnp.float32
    zw = (jnp.dot(h1, ww_ref[0:HID], preferred_element_type=f32)
          + jnp.dot(h2, ww_ref[HID:2 * HID], preferred_element_type=f32)
          + jnp.dot(h3, ww_ref[2 * HID:], preferred_element_type=f32)
          + bw_ref[...])
    zp = (jnp.dot(h1, wp_ref[0:HID], preferred_element_type=f32)
          + jnp.dot(h2, wp_ref[HID:2 * HID], preferred_element_type=f32)
          + jnp.dot(h3, wp_ref[2 * HID:], preferred_element_type=f32)
          + bp_ref[...])
    zw_ref[...] = zw
    zp_ref[...] = zp


def _tc_rowsum_body(prod_ref, out_ref):
    out_ref[...] = jnp.sum(prod_ref[...], axis=1, keepdims=True)


def _rb_spec(nd=2):
    if nd == 2:
        return pl.BlockSpec((_RB, D), lambda i: (i, 0))
    return pl.BlockSpec((NC, _RB, D), lambda i: (0, i, 0))


_dinv_spec = pl.BlockSpec((_RB, 1), lambda i: (i, 0))
_bias_spec = pl.BlockSpec((1, HID), lambda i: (0, 0))


def kernel(x, edge_index, src_idx, tgt_idx, W1, b1, W2, b2, W3, b3, Ww, bw, Wp, bp):
    src = edge_index[0]
    dst = edge_index[1]
    pad = E_PAD - E
    ar = jnp.arange(pad, dtype=jnp.int32)
    src_p = jnp.concatenate([src, ar % N])
    dst_p = jnp.concatenate([dst, N + (ar % (NPAD - N))])

    degp = _make_deg_kernel()(dst_p)
    degp3 = degp[:, :N].reshape(NC, N, 1)

    dinv, y1 = pl.pallas_call(
        _tc_pre_body,
        grid=(_GRID,),
        in_specs=[pl.BlockSpec((NC, _RB, 1), lambda i: (0, i, 0)),
                  _rb_spec(), pl.BlockSpec((D, HID), lambda i: (0, 0))],
        out_specs=[_dinv_spec, _rb_spec()],
        out_shape=[jax.ShapeDtypeStruct((N, 1), jnp.float32),
                   jax.ShapeDtypeStruct((N, HID), jnp.float32)],
    )(degp3, x, W1)

    hs = []
    y = y1
    for (bcur, Wn) in ((b1, W2), (b2, W3)):
        accp = _make_msgpass_kernel()(y, src_p, dst_p)
        a = accp[:, :N, :]
        h, y = pl.pallas_call(
            _tc_layer_body,
            grid=(_GRID,),
            in_specs=[_rb_spec(3), _rb_spec(), _dinv_spec, _bias_spec,
                      pl.BlockSpec((HID, HID), lambda i: (0, 0))],
            out_specs=[_rb_spec(), _rb_spec()],
            out_shape=[jax.ShapeDtypeStruct((N, HID), jnp.float32),
                       jax.ShapeDtypeStruct((N, HID), jnp.float32)],
        )(a, y, dinv, bcur.reshape(1, HID), Wn)
        hs.append(h)

    accp = _make_msgpass_kernel()(y, src_p, dst_p)
    a = accp[:, :N, :]
    zw, zp = pl.pallas_call(
        _tc_post_body,
        grid=(_GRID,),
        in_specs=[_rb_spec(3), _rb_spec(), _dinv_spec, _bias_spec,
                  _rb_spec(), _rb_spec(),
                  pl.BlockSpec((3 * HID, HID), lambda i: (0, 0)), _bias_spec,
                  pl.BlockSpec((3 * HID, HID), lambda i: (0, 0)), _bias_spec],
        out_specs=[_rb_spec(), _rb_spec()],
        out_shape=[jax.ShapeDtypeStruct((N, HID), jnp.float32),
                   jax.ShapeDtypeStruct((N, HID), jnp.float32)],
    )(a, y, dinv, b3.reshape(1, HID), hs[0], hs[1],
      Ww, bw.reshape(1, HID), Wp, bp.reshape(1, HID))

    prod = _make_pairgather_kernel()(zw, zp, tgt_idx, src_idx)
    out = pl.pallas_call(
        _tc_rowsum_body,
        grid=(1,),
        in_specs=[pl.BlockSpec((B, D), lambda i: (0, 0))],
        out_specs=pl.BlockSpec((B, 1), lambda i: (0, 0)),
        out_shape=jax.ShapeDtypeStruct((B, 1), jnp.float32),
    )(prod)
    return out.reshape(B)


# speedup vs baseline: 2.9508x; 1.3173x over previous
import functools

import jax
import jax.numpy as jnp
from jax import lax
from jax.experimental import pallas as pl
from jax.experimental.pallas import tpu as pltpu
from jax.experimental.pallas import tpu_sc as plsc

N = 10000
E = 320000
D = 128
HID = 128
B = 1024

NC, NS, L = 2, 16, 16
NW = NC * NS
BLK = 128
NBLK = 80
EPW = NBLK * BLK
E_PAD = EPW * NW
NPAD = 10240
RPT = NPAD // NS
BATW = B // NW

def _wid():
    return lax.axis_index("s") * NC + lax.axis_index("c")


def _sc_mesh():
    return plsc.VectorSubcoreMesh(core_axis_name="c", subcore_axis_name="s",
                                  num_cores=NC, num_subcores=NS)


@functools.cache
def _make_deg_kernel():
    return pl.kernel(
        _deg_body,
        out_type=jax.ShapeDtypeStruct((NC, NPAD), jnp.float32),
        mesh=_sc_mesh(),
        scratch_types=[
            pltpu.VMEM((BLK,), jnp.int32),
            pltpu.VMEM((BLK,), jnp.float32),
            pltpu.VMEM((RPT,), jnp.float32),
            pltpu.VMEM_SHARED((NPAD,), jnp.float32),
        ],
    )


def _deg_body(dst_hbm, degp_hbm, didx_v, ones_v, zbuf_v, acc_sh):
    cid = lax.axis_index("c")
    sid = lax.axis_index("s")
    wid = _wid()

    one16 = jnp.full((L,), 1.0, jnp.float32)
    zero16 = jnp.zeros((L,), jnp.float32)

    def fill_ones(i, _):
        ones_v[pl.ds(i * L, L)] = one16
        return 0

    lax.fori_loop(0, BLK // L, fill_ones, 0)

    def fill_zero(i, _):
        zbuf_v[pl.ds(i * L, L)] = zero16
        return 0

    lax.fori_loop(0, RPT // L, fill_zero, 0)
    pltpu.sync_copy(zbuf_v, acc_sh.at[pl.ds(sid * RPT, RPT)])
    plsc.subcore_barrier()

    base = wid * EPW

    def body(j, _):
        pltpu.sync_copy(dst_hbm.at[pl.ds(base + j * BLK, BLK)], didx_v)
        pltpu.sync_copy(ones_v, acc_sh.at[didx_v], add=True)
        return 0

    lax.fori_loop(0, NBLK, body, 0)
    plsc.subcore_barrier()
    pltpu.sync_copy(acc_sh.at[pl.ds(sid * RPT, RPT)],
                    degp_hbm.at[cid, pl.ds(sid * RPT, RPT)])


@functools.cache
def _make_msgpass_kernel():
    return pl.kernel(
        _msgpass_body,
        out_type=jax.ShapeDtypeStruct((NC, NPAD, D), jnp.float32),
        mesh=_sc_mesh(),
        scratch_types=[
            pltpu.VMEM((BLK,), jnp.int32),
            pltpu.VMEM((BLK,), jnp.int32),
            pltpu.VMEM((BLK,), jnp.int32),
            pltpu.VMEM((BLK,), jnp.int32),
            pltpu.VMEM((BLK, D), jnp.float32),
            pltpu.VMEM((BLK, D), jnp.float32),
            pltpu.SemaphoreType.DMA,
            pltpu.SemaphoreType.DMA,
            pltpu.VMEM_SHARED((NPAD, D), jnp.float32),
        ],
    )


def _msgpass_body(y_hbm, src_hbm, dst_hbm, accp_hbm,
                  sidx0_v, didx0_v, sidx1_v, didx1_v,
                  buf0_v, buf1_v, sem0, sem1, acc_sh):
    cid = lax.axis_index("c")
    sid = lax.axis_index("s")
    wid = _wid()

    zero16 = jnp.zeros((L,), jnp.float32)

    def fill_zero(i, _):
        for k in range(D // L):
            buf0_v[i, pl.ds(k * L, L)] = zero16
        return 0

    lax.fori_loop(0, BLK, fill_zero, 0)
    for k in range(RPT // BLK):
        pltpu.sync_copy(buf0_v, acc_sh.at[pl.ds(sid * RPT + k * BLK, BLK)])
    plsc.subcore_barrier()

    base = wid * EPW

    def body(g, _):
        j0 = 2 * g
        pltpu.sync_copy(src_hbm.at[pl.ds(base + j0 * BLK, BLK)], sidx0_v)
        pltpu.sync_copy(dst_hbm.at[pl.ds(base + j0 * BLK, BLK)], didx0_v)
        d0 = pltpu.async_copy(y_hbm.at[sidx0_v], buf0_v, sem0)
        pltpu.sync_copy(src_hbm.at[pl.ds(base + (j0 + 1) * BLK, BLK)], sidx1_v)
        pltpu.sync_copy(dst_hbm.at[pl.ds(base + (j0 + 1) * BLK, BLK)], didx1_v)
        d1 = pltpu.async_copy(y_hbm.at[sidx1_v], buf1_v, sem1)
        d0.wait()
        pltpu.sync_copy(buf0_v, acc_sh.at[didx0_v], add=True)
        d1.wait()
        pltpu.sync_copy(buf1_v, acc_sh.at[didx1_v], add=True)
        return 0

    lax.fori_loop(0, NBLK // 2, body, 0)
    plsc.subcore_barrier()
    for k in range(RPT // BLK):
        r0 = sid * RPT + k * BLK
        pltpu.sync_copy(acc_sh.at[pl.ds(r0, BLK)],
                        accp_hbm.at[cid, pl.ds(r0, BLK)])


@functools.cache
def _make_pairgather_kernel():
    return pl.kernel(
        _pairgather_body,
        out_type=jax.ShapeDtypeStruct((B, D), jnp.float32),
        mesh=_sc_mesh(),
        scratch_types=[
            pltpu.VMEM((BATW,), jnp.int32),
            pltpu.VMEM((BATW,), jnp.int32),
            pltpu.VMEM((BATW, D), jnp.float32),
            pltpu.VMEM((BATW, D), jnp.float32),
        ],
    )


def _pairgather_body(zw_hbm, zp_hbm, tgt_hbm, srcb_hbm, prod_hbm,
                     tidx_v, sidx_v, bufw_v, bufp_v):
    wid = _wid()
    base = wid * BATW
    pltpu.sync_copy(tgt_hbm.at[pl.ds(base, BATW)], tidx_v)
    pltpu.sync_copy(srcb_hbm.at[pl.ds(base, BATW)], sidx_v)
    pltpu.sync_copy(zw_hbm.at[tidx_v], bufw_v)
    pltpu.sync_copy(zp_hbm.at[sidx_v], bufp_v)

    def mul_row(i, _):
        for k in range(D // L):
            s = pl.ds(k * L, L)
            bufw_v[i, s] = bufw_v[i, s] * bufp_v[i, s]
        return 0

    lax.fori_loop(0, BATW, mul_row, 0)
    pltpu.sync_copy(bufw_v, prod_hbm.at[pl.ds(base, BATW)])


_RB = 1000
_GRID = N // _RB


def _tc_pre_body(degp_ref, x_ref, w1_ref, dinv_ref, y1_ref):
    dp = degp_ref[0] + degp_ref[1] + 1.0
    dinv = lax.rsqrt(jnp.maximum(dp, 1e-12))
    dinv_ref[...] = dinv
    y = jnp.dot(x_ref[...], w1_ref[...], preferred_element_type=jnp.float32)
    y1_ref[...] = y * dinv


def _tc_layer_body(a_ref, y_ref, dinv_ref, b_ref, wn_ref, h_ref, yn_ref):
    dinv = dinv_ref[...]
    h = jnp.maximum(dinv * (a_ref[0] + a_ref[1] + y_ref[...]) + b_ref[...], 0.0)
    h_ref[...] = h
    yn = jnp.dot(h, wn_ref[...], preferred_element_type=jnp.float32)
    yn_ref[...] = yn * dinv


def _tc_post_body(a_ref, y_ref, dinv_ref, b3_ref, h1_ref, h2_ref,
                  ww_ref, bw_ref, wp_ref, bp_ref, zw_ref, zp_ref):
    dinv = dinv_ref[...]
    h3 = jnp.maximum(dinv * (a_ref[0] + a_ref[1] + y_ref[...]) + b3_ref[...], 0.0)
    h1 = h1_ref[...]
    h2 = h2_ref[...]
    f32 = jnp.float32
    zw = (jnp.dot(h1, ww_ref[0:HID], preferred_element_type=f32)
          + jnp.dot(h2, ww_ref[HID:2 * HID], preferred_element_type=f32)
          + jnp.dot(h3, ww_ref[2 * HID:], preferred_element_type=f32)
          + bw_ref[...])
    zp = (jnp.dot(h1, wp_ref[0:HID], preferred_element_type=f32)
          + jnp.dot(h2, wp_ref[HID:2 * HID], preferred_element_type=f32)
          + jnp.dot(h3, wp_ref[2 * HID:], preferred_element_type=f32)
          + bp_ref[...])
    zw_ref[...] = zw
    zp_ref[...] = zp


def _tc_rowsum_body(prod_ref, out_ref):
    out_ref[...] = jnp.sum(prod_ref[...], axis=1, keepdims=True)


def _rb_spec(nd=2):
    if nd == 2:
        return pl.BlockSpec((_RB, D), lambda i: (i, 0))
    return pl.BlockSpec((NC, _RB, D), lambda i: (0, i, 0))


_dinv_spec = pl.BlockSpec((_RB, 1), lambda i: (i, 0))
_bias_spec = pl.BlockSpec((1, HID), lambda i: (0, 0))


def kernel(x, edge_index, src_idx, tgt_idx, W1, b1, W2, b2, W3, b3, Ww, bw, Wp, bp):
    src = edge_index[0]
    dst = edge_index[1]
    pad = E_PAD - E
    ar = jnp.arange(pad, dtype=jnp.int32)
    src_p = jnp.concatenate([src, ar % N])
    dst_p = jnp.concatenate([dst, N + (ar % (NPAD - N))])

    degp = _make_deg_kernel()(dst_p)
    degp3 = degp[:, :N].reshape(NC, N, 1)

    dinv, y1 = pl.pallas_call(
        _tc_pre_body,
        grid=(_GRID,),
        in_specs=[pl.BlockSpec((NC, _RB, 1), lambda i: (0, i, 0)),
                  _rb_spec(), pl.BlockSpec((D, HID), lambda i: (0, 0))],
        out_specs=[_dinv_spec, _rb_spec()],
        out_shape=[jax.ShapeDtypeStruct((N, 1), jnp.float32),
                   jax.ShapeDtypeStruct((N, HID), jnp.float32)],
    )(degp3, x, W1)

    hs = []
    y = y1
    for (bcur, Wn) in ((b1, W2), (b2, W3)):
        accp = _make_msgpass_kernel()(y, src_p, dst_p)
        a = accp[:, :N, :]
        h, y = pl.pallas_call(
            _tc_layer_body,
            grid=(_GRID,),
            in_specs=[_rb_spec(3), _rb_spec(), _dinv_spec, _bias_spec,
                      pl.BlockSpec((HID, HID), lambda i: (0, 0))],
            out_specs=[_rb_spec(), _rb_spec()],
            out_shape=[jax.ShapeDtypeStruct((N, HID), jnp.float32),
                       jax.ShapeDtypeStruct((N, HID), jnp.float32)],
        )(a, y, dinv, bcur.reshape(1, HID), Wn)
        hs.append(h)

    accp = _make_msgpass_kernel()(y, src_p, dst_p)
    a = accp[:, :N, :]
    zw, zp = pl.pallas_call(
        _tc_post_body,
        grid=(_GRID,),
        in_specs=[_rb_spec(3), _rb_spec(), _dinv_spec, _bias_spec,
                  _rb_spec(), _rb_spec(),
                  pl.BlockSpec((3 * HID, HID), lambda i: (0, 0)), _bias_spec,
                  pl.BlockSpec((3 * HID, HID), lambda i: (0, 0)), _bias_spec],
        out_specs=[_rb_spec(), _rb_spec()],
        out_shape=[jax.ShapeDtypeStruct((N, HID), jnp.float32),
                   jax.ShapeDtypeStruct((N, HID), jnp.float32)],
    )(a, y, dinv, b3.reshape(1, HID), hs[0], hs[1],
      Ww, bw.reshape(1, HID), Wp, bp.reshape(1, HID))

    prod = _make_pairgather_kernel()(zw, zp, tgt_idx, src_idx)
    out = pl.pallas_call(
        _tc_rowsum_body,
        grid=(1,),
        in_specs=[pl.BlockSpec((B, D), lambda i: (0, 0))],
        out_specs=pl.BlockSpec((B, 1), lambda i: (0, 0)),
        out_shape=jax.ShapeDtypeStruct((B, 1), jnp.float32),
    )(prod)
    return out.reshape(B)


# speedup vs baseline: 2.9518x; 1.0003x over previous
import functools

import jax
import jax.numpy as jnp
from jax import lax
from jax.experimental import pallas as pl
from jax.experimental.pallas import tpu as pltpu
from jax.experimental.pallas import tpu_sc as plsc

N = 10000
E = 320000
D = 128
HID = 128
B = 1024

NC, NS, L = 2, 16, 16
NW = NC * NS
BLK = 128
NBLK = 80
EPW = NBLK * BLK
E_PAD = EPW * NW
NPAD = 10240
RPT = NPAD // NS
BATW = B // NW

def _wid():
    return lax.axis_index("s") * NC + lax.axis_index("c")


def _sc_mesh():
    return plsc.VectorSubcoreMesh(core_axis_name="c", subcore_axis_name="s",
                                  num_cores=NC, num_subcores=NS)


@functools.cache
def _make_deg_kernel():
    return pl.kernel(
        _deg_body,
        out_type=jax.ShapeDtypeStruct((NC, NPAD), jnp.float32),
        mesh=_sc_mesh(),
        scratch_types=[
            pltpu.VMEM((BLK,), jnp.int32),
            pltpu.VMEM((BLK,), jnp.float32),
            pltpu.VMEM((RPT,), jnp.float32),
            pltpu.VMEM_SHARED((NPAD,), jnp.float32),
        ],
    )


def _deg_body(dst_hbm, degp_hbm, didx_v, ones_v, zbuf_v, acc_sh):
    cid = lax.axis_index("c")
    sid = lax.axis_index("s")
    wid = _wid()

    one16 = jnp.full((L,), 1.0, jnp.float32)
    zero16 = jnp.zeros((L,), jnp.float32)

    def fill_ones(i, _):
        ones_v[pl.ds(i * L, L)] = one16
        return 0

    lax.fori_loop(0, BLK // L, fill_ones, 0)

    def fill_zero(i, _):
        zbuf_v[pl.ds(i * L, L)] = zero16
        return 0

    lax.fori_loop(0, RPT // L, fill_zero, 0)
    pltpu.sync_copy(zbuf_v, acc_sh.at[pl.ds(sid * RPT, RPT)])
    plsc.subcore_barrier()

    base = wid * EPW

    def body(j, _):
        pltpu.sync_copy(dst_hbm.at[pl.ds(base + j * BLK, BLK)], didx_v)
        pltpu.sync_copy(ones_v, acc_sh.at[didx_v], add=True)
        return 0

    lax.fori_loop(0, NBLK, body, 0)
    plsc.subcore_barrier()
    pltpu.sync_copy(acc_sh.at[pl.ds(sid * RPT, RPT)],
                    degp_hbm.at[cid, pl.ds(sid * RPT, RPT)])


@functools.cache
def _make_msgpass_kernel():
    return pl.kernel(
        _msgpass_body,
        out_type=jax.ShapeDtypeStruct((NC, NPAD, D), jnp.float32),
        mesh=_sc_mesh(),
        scratch_types=[
            pltpu.VMEM((BLK,), jnp.int32),
            pltpu.VMEM((BLK,), jnp.int32),
            pltpu.VMEM((BLK,), jnp.int32),
            pltpu.VMEM((BLK,), jnp.int32),
            pltpu.VMEM((BLK, D), jnp.float32),
            pltpu.VMEM((BLK, D), jnp.float32),
            pltpu.SemaphoreType.DMA,
            pltpu.SemaphoreType.DMA,
            pltpu.SemaphoreType.DMA,
            pltpu.SemaphoreType.DMA,
            pltpu.VMEM_SHARED((NPAD, D), jnp.float32),
        ],
    )


def _msgpass_body(y_hbm, src_hbm, dst_hbm, accp_hbm,
                  sidx0_v, didx0_v, sidx1_v, didx1_v,
                  buf0_v, buf1_v, sem0, sem1, sem2, sem3, acc_sh):
    cid = lax.axis_index("c")
    sid = lax.axis_index("s")
    wid = _wid()

    zero16 = jnp.zeros((L,), jnp.float32)

    def fill_zero(i, _):
        for k in range(D // L):
            buf0_v[i, pl.ds(k * L, L)] = zero16
        return 0

    lax.fori_loop(0, BLK, fill_zero, 0)
    for k in range(RPT // BLK):
        pltpu.sync_copy(buf0_v, acc_sh.at[pl.ds(sid * RPT + k * BLK, BLK)])
    plsc.subcore_barrier()

    base = wid * EPW

    def body(g, _):
        j0 = 2 * g
        pltpu.sync_copy(src_hbm.at[pl.ds(base + j0 * BLK, BLK)], sidx0_v)
        pltpu.sync_copy(dst_hbm.at[pl.ds(base + j0 * BLK, BLK)], didx0_v)
        d0 = pltpu.async_copy(y_hbm.at[sidx0_v], buf0_v, sem0)
        pltpu.sync_copy(src_hbm.at[pl.ds(base + (j0 + 1) * BLK, BLK)], sidx1_v)
        pltpu.sync_copy(dst_hbm.at[pl.ds(base + (j0 + 1) * BLK, BLK)], didx1_v)
        d1 = pltpu.async_copy(y_hbm.at[sidx1_v], buf1_v, sem1)
        d0.wait()
        s0 = pltpu.async_copy(buf0_v, acc_sh.at[didx0_v], sem2, add=True)
        d1.wait()
        s1 = pltpu.async_copy(buf1_v, acc_sh.at[didx1_v], sem3, add=True)
        s0.wait()
        s1.wait()
        return 0

    lax.fori_loop(0, NBLK // 2, body, 0)
    plsc.subcore_barrier()
    for k in range(RPT // BLK):
        r0 = sid * RPT + k * BLK
        pltpu.sync_copy(acc_sh.at[pl.ds(r0, BLK)],
                        accp_hbm.at[cid, pl.ds(r0, BLK)])


@functools.cache
def _make_pairgather_kernel():
    return pl.kernel(
        _pairgather_body,
        out_type=jax.ShapeDtypeStruct((B, D), jnp.float32),
        mesh=_sc_mesh(),
        scratch_types=[
            pltpu.VMEM((BATW,), jnp.int32),
            pltpu.VMEM((BATW,), jnp.int32),
            pltpu.VMEM((BATW, D), jnp.float32),
            pltpu.VMEM((BATW, D), jnp.float32),
        ],
    )


def _pairgather_body(zw_hbm, zp_hbm, tgt_hbm, srcb_hbm, prod_hbm,
                     tidx_v, sidx_v, bufw_v, bufp_v):
    wid = _wid()
    base = wid * BATW
    pltpu.sync_copy(tgt_hbm.at[pl.ds(base, BATW)], tidx_v)
    pltpu.sync_copy(srcb_hbm.at[pl.ds(base, BATW)], sidx_v)
    pltpu.sync_copy(zw_hbm.at[tidx_v], bufw_v)
    pltpu.sync_copy(zp_hbm.at[sidx_v], bufp_v)

    def mul_row(i, _):
        for k in range(D // L):
            s = pl.ds(k * L, L)
            bufw_v[i, s] = bufw_v[i, s] * bufp_v[i, s]
        return 0

    lax.fori_loop(0, BATW, mul_row, 0)
    pltpu.sync_copy(bufw_v, prod_hbm.at[pl.ds(base, BATW)])


_RB = 1000
_GRID = N // _RB


def _tc_pre_body(degp_ref, x_ref, w1_ref, dinv_ref, y1_ref):
    dp = degp_ref[0] + degp_ref[1] + 1.0
    dinv = lax.rsqrt(jnp.maximum(dp, 1e-12))
    dinv_ref[...] = dinv
    y = jnp.dot(x_ref[...], w1_ref[...], preferred_element_type=jnp.float32)
    y1_ref[...] = y * dinv


def _tc_layer_body(a_ref, y_ref, dinv_ref, b_ref, wn_ref, h_ref, yn_ref):
    dinv = dinv_ref[...]
    h = jnp.maximum(dinv * (a_ref[0] + a_ref[1] + y_ref[...]) + b_ref[...], 0.0)
    h_ref[...] = h
    yn = jnp.dot(h, wn_ref[...], preferred_element_type=jnp.float32)
    yn_ref[...] = yn * dinv


def _tc_post_body(a_ref, y_ref, dinv_ref, b3_ref, h1_ref, h2_ref,
                  ww_ref, bw_ref, wp_ref, bp_ref, zw_ref, zp_ref):
    dinv = dinv_ref[...]
    h3 = jnp.maximum(dinv * (a_ref[0] + a_ref[1] + y_ref[...]) + b3_ref[...], 0.0)
    h1 = h1_ref[...]
    h2 = h2_ref[...]
    f32 = jnp.float32
    zw = (jnp.dot(h1, ww_ref[0:HID], preferred_element_type=f32)
          + jnp.dot(h2, ww_ref[HID:2 * HID], preferred_element_type=f32)
          + jnp.dot(h3, ww_ref[2 * HID:], preferred_element_type=f32)
          + bw_ref[...])
    zp = (jnp.dot(h1, wp_ref[0:HID], preferred_element_type=f32)
          + jnp.dot(h2, wp_ref[HID:2 * HID], preferred_element_type=f32)
          + jnp.dot(h3, wp_ref[2 * HID:], preferred_element_type=f32)
          + bp_ref[...])
    zw_ref[...] = zw
    zp_ref[...] = zp


def _tc_rowsum_body(prod_ref, out_ref):
    out_ref[...] = jnp.sum(prod_ref[...], axis=1, keepdims=True)


def _rb_spec(nd=2):
    if nd == 2:
        return pl.BlockSpec((_RB, D), lambda i: (i, 0))
    return pl.BlockSpec((NC, _RB, D), lambda i: (0, i, 0))


_dinv_spec = pl.BlockSpec((_RB, 1), lambda i: (i, 0))
_bias_spec = pl.BlockSpec((1, HID), lambda i: (0, 0))


def kernel(x, edge_index, src_idx, tgt_idx, W1, b1, W2, b2, W3, b3, Ww, bw, Wp, bp):
    src = edge_index[0]
    dst = edge_index[1]
    pad = E_PAD - E
    ar = jnp.arange(pad, dtype=jnp.int32)
    src_p = jnp.concatenate([src, ar % N])
    dst_p = jnp.concatenate([dst, N + (ar % (NPAD - N))])

    degp = _make_deg_kernel()(dst_p)
    degp3 = degp[:, :N].reshape(NC, N, 1)

    dinv, y1 = pl.pallas_call(
        _tc_pre_body,
        grid=(_GRID,),
        in_specs=[pl.BlockSpec((NC, _RB, 1), lambda i: (0, i, 0)),
                  _rb_spec(), pl.BlockSpec((D, HID), lambda i: (0, 0))],
        out_specs=[_dinv_spec, _rb_spec()],
        out_shape=[jax.ShapeDtypeStruct((N, 1), jnp.float32),
                   jax.ShapeDtypeStruct((N, HID), jnp.float32)],
    )(degp3, x, W1)

    hs = []
    y = y1
    for (bcur, Wn) in ((b1, W2), (b2, W3)):
        accp = _make_msgpass_kernel()(y, src_p, dst_p)
        a = accp[:, :N, :]
        h, y = pl.pallas_call(
            _tc_layer_body,
            grid=(_GRID,),
            in_specs=[_rb_spec(3), _rb_spec(), _dinv_spec, _bias_spec,
                      pl.BlockSpec((HID, HID), lambda i: (0, 0))],
            out_specs=[_rb_spec(), _rb_spec()],
            out_shape=[jax.ShapeDtypeStruct((N, HID), jnp.float32),
                       jax.ShapeDtypeStruct((N, HID), jnp.float32)],
        )(a, y, dinv, bcur.reshape(1, HID), Wn)
        hs.append(h)

    accp = _make_msgpass_kernel()(y, src_p, dst_p)
    a = accp[:, :N, :]
    zw, zp = pl.pallas_call(
        _tc_post_body,
        grid=(_GRID,),
        in_specs=[_rb_spec(3), _rb_spec(), _dinv_spec, _bias_spec,
                  _rb_spec(), _rb_spec(),
                  pl.BlockSpec((3 * HID, HID), lambda i: (0, 0)), _bias_spec,
                  pl.BlockSpec((3 * HID, HID), lambda i: (0, 0)), _bias_spec],
        out_specs=[_rb_spec(), _rb_spec()],
        out_shape=[jax.ShapeDtypeStruct((N, HID), jnp.float32),
                   jax.ShapeDtypeStruct((N, HID), jnp.float32)],
    )(a, y, dinv, b3.reshape(1, HID), hs[0], hs[1],
      Ww, bw.reshape(1, HID), Wp, bp.reshape(1, HID))

    prod = _make_pairgather_kernel()(zw, zp, tgt_idx, src_idx)
    out = pl.pallas_call(
        _tc_rowsum_body,
        grid=(1,),
        in_specs=[pl.BlockSpec((B, D), lambda i: (0, 0))],
        out_specs=pl.BlockSpec((B, 1), lambda i: (0, 0)),
        out_shape=jax.ShapeDtypeStruct((B, 1), jnp.float32),
    )(prod)
    return out.reshape(B)


# speedup vs baseline: 3.6502x; 1.2366x over previous
import functools

import jax
import jax.numpy as jnp
from jax import lax
from jax.experimental import pallas as pl
from jax.experimental.pallas import tpu as pltpu
from jax.experimental.pallas import tpu_sc as plsc

N = 10000
E = 320000
D = 128
HID = 128
B = 1024

NC, NS, L = 2, 16, 16
NW = NC * NS
BLK = 128
NBLK = 80
EPW = NBLK * BLK
E_PAD = EPW * NW
NPAD = 10240
RPT = NPAD // NS
BATW = B // NW
G = 8
NGRP = NBLK // G

def _wid():
    return lax.axis_index("s") * NC + lax.axis_index("c")


def _sc_mesh():
    return plsc.VectorSubcoreMesh(core_axis_name="c", subcore_axis_name="s",
                                  num_cores=NC, num_subcores=NS)


@functools.cache
def _make_deg_kernel():
    return pl.kernel(
        _deg_body,
        out_type=jax.ShapeDtypeStruct((NC, NPAD), jnp.float32),
        mesh=_sc_mesh(),
        scratch_types=[
            pltpu.VMEM((BLK,), jnp.int32),
            pltpu.VMEM((BLK,), jnp.float32),
            pltpu.VMEM((RPT,), jnp.float32),
            pltpu.VMEM_SHARED((NPAD,), jnp.float32),
        ],
    )


def _deg_body(dst_hbm, degp_hbm, didx_v, ones_v, zbuf_v, acc_sh):
    cid = lax.axis_index("c")
    sid = lax.axis_index("s")
    wid = _wid()

    one16 = jnp.full((L,), 1.0, jnp.float32)
    zero16 = jnp.zeros((L,), jnp.float32)

    def fill_ones(i, _):
        ones_v[pl.ds(i * L, L)] = one16
        return 0

    lax.fori_loop(0, BLK // L, fill_ones, 0)

    def fill_zero(i, _):
        zbuf_v[pl.ds(i * L, L)] = zero16
        return 0

    lax.fori_loop(0, RPT // L, fill_zero, 0)
    pltpu.sync_copy(zbuf_v, acc_sh.at[pl.ds(sid * RPT, RPT)])
    plsc.subcore_barrier()

    base = wid * EPW

    def body(j, _):
        pltpu.sync_copy(dst_hbm.at[pl.ds(base + j * BLK, BLK)], didx_v)
        pltpu.sync_copy(ones_v, acc_sh.at[didx_v], add=True)
        return 0

    lax.fori_loop(0, NBLK, body, 0)
    plsc.subcore_barrier()
    pltpu.sync_copy(acc_sh.at[pl.ds(sid * RPT, RPT)],
                    degp_hbm.at[cid, pl.ds(sid * RPT, RPT)])


@functools.cache
def _make_msgpass_kernel():
    return pl.kernel(
        _msgpass_body,
        out_type=jax.ShapeDtypeStruct((NC, NPAD, D), jnp.float32),
        mesh=_sc_mesh(),
        scratch_types=[
            pltpu.VMEM((G, BLK), jnp.int32),
            pltpu.VMEM((G, BLK), jnp.int32),
            pltpu.VMEM((BLK, D), jnp.float32),
            pltpu.VMEM((BLK, D), jnp.float32),
            pltpu.SemaphoreType.DMA,
            pltpu.SemaphoreType.DMA,
            pltpu.VMEM_SHARED((NPAD, D), jnp.float32),
        ],
    )


def _msgpass_body(y_hbm, src4_hbm, dst4_hbm, accp_hbm,
                  sg_v, dg_v, buf0_v, buf1_v, sem0, sem1, acc_sh):
    cid = lax.axis_index("c")
    sid = lax.axis_index("s")
    wid = _wid()

    zero16 = jnp.zeros((L,), jnp.float32)

    def fill_zero(i, _):
        for k in range(D // L):
            buf0_v[i, pl.ds(k * L, L)] = zero16
        return 0

    lax.fori_loop(0, BLK, fill_zero, 0)
    for k in range(RPT // BLK):
        pltpu.sync_copy(buf0_v, acc_sh.at[pl.ds(sid * RPT + k * BLK, BLK)])
    plsc.subcore_barrier()

    bufs = (buf0_v, buf1_v)
    sems = (sem0, sem1)

    def body(t, _):
        pltpu.sync_copy(src4_hbm.at[wid, t], sg_v)
        pltpu.sync_copy(dst4_hbm.at[wid, t], dg_v)
        d = [None, None]
        d[0] = pltpu.async_copy(y_hbm.at[sg_v.at[0]], buf0_v, sem0)
        for u in range(G):
            if u + 1 < G:
                d[(u + 1) % 2] = pltpu.async_copy(
                    y_hbm.at[sg_v.at[u + 1]], bufs[(u + 1) % 2],
                    sems[(u + 1) % 2])
            d[u % 2].wait()
            pltpu.sync_copy(bufs[u % 2], acc_sh.at[dg_v.at[u]], add=True)
        return 0

    lax.fori_loop(0, NGRP, body, 0)
    plsc.subcore_barrier()
    for k in range(RPT // BLK):
        r0 = sid * RPT + k * BLK
        pltpu.sync_copy(acc_sh.at[pl.ds(r0, BLK)],
                        accp_hbm.at[cid, pl.ds(r0, BLK)])


@functools.cache
def _make_pairgather_kernel():
    return pl.kernel(
        _pairgather_body,
        out_type=jax.ShapeDtypeStruct((B, D), jnp.float32),
        mesh=_sc_mesh(),
        scratch_types=[
            pltpu.VMEM((BATW,), jnp.int32),
            pltpu.VMEM((BATW,), jnp.int32),
            pltpu.VMEM((BATW, D), jnp.float32),
            pltpu.VMEM((BATW, D), jnp.float32),
        ],
    )


def _pairgather_body(zw_hbm, zp_hbm, tgt_hbm, srcb_hbm, prod_hbm,
                     tidx_v, sidx_v, bufw_v, bufp_v):
    wid = _wid()
    base = wid * BATW
    pltpu.sync_copy(tgt_hbm.at[pl.ds(base, BATW)], tidx_v)
    pltpu.sync_copy(srcb_hbm.at[pl.ds(base, BATW)], sidx_v)
    pltpu.sync_copy(zw_hbm.at[tidx_v], bufw_v)
    pltpu.sync_copy(zp_hbm.at[sidx_v], bufp_v)

    def mul_row(i, _):
        for k in range(D // L):
            s = pl.ds(k * L, L)
            bufw_v[i, s] = bufw_v[i, s] * bufp_v[i, s]
        return 0

    lax.fori_loop(0, BATW, mul_row, 0)
    pltpu.sync_copy(bufw_v, prod_hbm.at[pl.ds(base, BATW)])


_RB = 1000
_GRID = N // _RB


def _tc_pre_body(degp_ref, x_ref, w1_ref, dinv_ref, y1_ref):
    dp = degp_ref[0] + degp_ref[1] + 1.0
    dinv = lax.rsqrt(jnp.maximum(dp, 1e-12))
    dinv_ref[...] = dinv
    y = jnp.dot(x_ref[...], w1_ref[...], preferred_element_type=jnp.float32)
    y1_ref[...] = y * dinv


def _tc_layer_body(a_ref, y_ref, dinv_ref, b_ref, wn_ref, h_ref, yn_ref):
    dinv = dinv_ref[...]
    h = jnp.maximum(dinv * (a_ref[0] + a_ref[1] + y_ref[...]) + b_ref[...], 0.0)
    h_ref[...] = h
    yn = jnp.dot(h, wn_ref[...], preferred_element_type=jnp.float32)
    yn_ref[...] = yn * dinv


def _tc_post_body(a_ref, y_ref, dinv_ref, b3_ref, h1_ref, h2_ref,
                  ww_ref, bw_ref, wp_ref, bp_ref, zw_ref, zp_ref):
    dinv = dinv_ref[...]
    h3 = jnp.maximum(dinv * (a_ref[0] + a_ref[1] + y_ref[...]) + b3_ref[...], 0.0)
    h1 = h1_ref[...]
    h2 = h2_ref[...]
    f32 = jnp.float32
    zw = (jnp.dot(h1, ww_ref[0:HID], preferred_element_type=f32)
          + jnp.dot(h2, ww_ref[HID:2 * HID], preferred_element_type=f32)
          + jnp.dot(h3, ww_ref[2 * HID:], preferred_element_type=f32)
          + bw_ref[...])
    zp = (jnp.dot(h1, wp_ref[0:HID], preferred_element_type=f32)
          + jnp.dot(h2, wp_ref[HID:2 * HID], preferred_element_type=f32)
          + jnp.dot(h3, wp_ref[2 * HID:], preferred_element_type=f32)
          + bp_ref[...])
    zw_ref[...] = zw
    zp_ref[...] = zp


def _tc_rowsum_body(prod_ref, out_ref):
    out_ref[...] = jnp.sum(prod_ref[...], axis=1, keepdims=True)


def _rb_spec(nd=2):
    if nd == 2:
        return pl.BlockSpec((_RB, D), lambda i: (i, 0))
    return pl.BlockSpec((NC, _RB, D), lambda i: (0, i, 0))


_dinv_spec = pl.BlockSpec((_RB, 1), lambda i: (i, 0))
_bias_spec = pl.BlockSpec((1, HID), lambda i: (0, 0))


def kernel(x, edge_index, src_idx, tgt_idx, W1, b1, W2, b2, W3, b3, Ww, bw, Wp, bp):
    src = edge_index[0]
    dst = edge_index[1]
    pad = E_PAD - E
    ar = jnp.arange(pad, dtype=jnp.int32)
    src_p = jnp.concatenate([src, ar % N])
    dst_p = jnp.concatenate([dst, N + (ar % (NPAD - N))])
    src4 = src_p.reshape(NW, NGRP, G, BLK)
    dst4 = dst_p.reshape(NW, NGRP, G, BLK)

    degp = _make_deg_kernel()(dst_p)
    degp3 = degp[:, :N].reshape(NC, N, 1)

    dinv, y1 = pl.pallas_call(
        _tc_pre_body,
        grid=(_GRID,),
        in_specs=[pl.BlockSpec((NC, _RB, 1), lambda i: (0, i, 0)),
                  _rb_spec(), pl.BlockSpec((D, HID), lambda i: (0, 0))],
        out_specs=[_dinv_spec, _rb_spec()],
        out_shape=[jax.ShapeDtypeStruct((N, 1), jnp.float32),
                   jax.ShapeDtypeStruct((N, HID), jnp.float32)],
    )(degp3, x, W1)

    hs = []
    y = y1
    for (bcur, Wn) in ((b1, W2), (b2, W3)):
        accp = _make_msgpass_kernel()(y, src4, dst4)
        a = accp[:, :N, :]
        h, y = pl.pallas_call(
            _tc_layer_body,
            grid=(_GRID,),
            in_specs=[_rb_spec(3), _rb_spec(), _dinv_spec, _bias_spec,
                      pl.BlockSpec((HID, HID), lambda i: (0, 0))],
            out_specs=[_rb_spec(), _rb_spec()],
            out_shape=[jax.ShapeDtypeStruct((N, HID), jnp.float32),
                       jax.ShapeDtypeStruct((N, HID), jnp.float32)],
        )(a, y, dinv, bcur.reshape(1, HID), Wn)
        hs.append(h)

    accp = _make_msgpass_kernel()(y, src4, dst4)
    a = accp[:, :N, :]
    zw, zp = pl.pallas_call(
        _tc_post_body,
        grid=(_GRID,),
        in_specs=[_rb_spec(3), _rb_spec(), _dinv_spec, _bias_spec,
                  _rb_spec(), _rb_spec(),
                  pl.BlockSpec((3 * HID, HID), lambda i: (0, 0)), _bias_spec,
                  pl.BlockSpec((3 * HID, HID), lambda i: (0, 0)), _bias_spec],
        out_specs=[_rb_spec(), _rb_spec()],
        out_shape=[jax.ShapeDtypeStruct((N, HID), jnp.float32),
                   jax.ShapeDtypeStruct((N, HID), jnp.float32)],
    )(a, y, dinv, b3.reshape(1, HID), hs[0], hs[1],
      Ww, bw.reshape(1, HID), Wp, bp.reshape(1, HID))

    prod = _make_pairgather_kernel()(zw, zp, tgt_idx, src_idx)
    out = pl.pallas_call(
        _tc_rowsum_body,
        grid=(1,),
        in_specs=[pl.BlockSpec((B, D), lambda i: (0, 0))],
        out_specs=pl.BlockSpec((B, 1), lambda i: (0, 0)),
        out_shape=jax.ShapeDtypeStruct((B, 1), jnp.float32),
    )(prod)
    return out.reshape(B)
